# Initial kernel scaffold; baseline (speedup 1.0000x reference)
import jax
import jax.numpy as jnp
from jax.experimental import pallas as pl


def kernel(indices, weight):
    raise NotImplementedError("write your pallas kernel here")


# speedup vs baseline: 1.5666x; 1.5666x over previous
import functools

import jax
import jax.numpy as jnp
from jax import lax
from jax.experimental import pallas as pl
from jax.experimental.pallas import tpu as pltpu
from jax.experimental.pallas import tpu_sc as plsc

BATCH = 16384
FIELDS = 26
EMBED_DIM = 32

NUM_CORES = 2
NUM_SUBCORES = 16
NUM_WORKERS = NUM_CORES * NUM_SUBCORES

TOTAL = BATCH * FIELDS
PER_WORKER = TOTAL // NUM_WORKERS
CHUNK = 128
GROUP = 13
ROWS_PER_GROUP = CHUNK * GROUP
NUM_GROUPS = PER_WORKER // ROWS_PER_GROUP
NUM_CHUNKS = PER_WORKER // CHUNK

assert PER_WORKER * NUM_WORKERS == TOTAL
assert ROWS_PER_GROUP * NUM_GROUPS == PER_WORKER


@functools.partial(
    pl.kernel,
    mesh=plsc.VectorSubcoreMesh(core_axis_name="c", subcore_axis_name="s"),
    out_type=jax.ShapeDtypeStruct((TOTAL, EMBED_DIM), jnp.float32),
    scratch_types=[
        pltpu.VMEM((NUM_CHUNKS, CHUNK), jnp.int32),
        pltpu.VMEM((ROWS_PER_GROUP, EMBED_DIM), jnp.float32),
        pltpu.SemaphoreType.DMA,
        pltpu.SemaphoreType.DMA,
    ],
    compiler_params=pltpu.CompilerParams(use_tc_tiling_on_sc=False),
)
def _gather_kernel(idx_hbm, table_hbm, out_hbm, idx_v, rows_v, gsem, osem):
    wid = lax.axis_index("s") * NUM_CORES + lax.axis_index("c")
    base = wid * PER_WORKER
    pltpu.sync_copy(idx_hbm.at[wid], idx_v)

    def group_body(g, _):
        copies = []
        for j in range(GROUP):
            c = pltpu.async_copy(
                table_hbm.at[idx_v.at[g * GROUP + j]],
                rows_v.at[pl.ds(j * CHUNK, CHUNK)],
                gsem,
            )
            copies.append(c)
        for c in copies:
            c.wait()
        pltpu.async_copy(
            rows_v,
            out_hbm.at[pl.ds(base + g * ROWS_PER_GROUP, ROWS_PER_GROUP)],
            osem,
        ).wait()
        return ()

    lax.fori_loop(0, NUM_GROUPS, group_body, ())


def kernel(indices, weight):
    idx = indices.reshape(NUM_WORKERS, NUM_CHUNKS, CHUNK).astype(jnp.int32)
    out = _gather_kernel(idx, weight)
    return out.reshape(BATCH, FIELDS, EMBED_DIM)


# speedup vs baseline: 2.0351x; 1.2990x over previous
import functools

import jax
import jax.numpy as jnp
from jax import lax
from jax.experimental import pallas as pl
from jax.experimental.pallas import tpu as pltpu
from jax.experimental.pallas import tpu_sc as plsc

BATCH = 16384
FIELDS = 26
EMBED_DIM = 32
NUM_NODES = 1000000

NUM_CORES = 2
NUM_SUBCORES = 16
NUM_WORKERS = NUM_CORES * NUM_SUBCORES

TOTAL = BATCH * FIELDS
PER_WORKER = TOTAL // NUM_WORKERS
CHUNK = 128
GROUP = 13
ROWS_PER_GROUP = CHUNK * GROUP
NUM_GROUPS = PER_WORKER // ROWS_PER_GROUP
NUM_CHUNKS = PER_WORKER // CHUNK

RCOLS = 1024
RGRID = 245
V_QUARTER = RCOLS * RGRID
NODES_LIN = 4 * V_QUARTER

assert PER_WORKER * NUM_WORKERS == TOTAL
assert ROWS_PER_GROUP * NUM_GROUPS == PER_WORKER


def _repack_block(w0, w1, w2, w3, out_ref):
    out_ref[:, 0:32] = jnp.transpose(w0[...])
    out_ref[:, 32:64] = jnp.transpose(w1[...])
    out_ref[:, 64:96] = jnp.transpose(w2[...])
    out_ref[:, 96:128] = jnp.transpose(w3[...])


def _repack(wt):
    max_blk = NUM_NODES // RCOLS
    specs = [
        pl.BlockSpec(
            (EMBED_DIM, RCOLS),
            functools.partial(
                lambda q, i: (0, jnp.minimum(q * RGRID + i, max_blk)), q
            ),
        )
        for q in range(4)
    ]
    return pl.pallas_call(
        _repack_block,
        grid=(RGRID,),
        in_specs=specs,
        out_specs=pl.BlockSpec((RCOLS, 128), lambda i: (i, 0)),
        out_shape=jax.ShapeDtypeStruct((V_QUARTER, 128), jnp.float32),
    )(wt, wt, wt, wt)


@functools.partial(
    pl.kernel,
    mesh=plsc.VectorSubcoreMesh(core_axis_name="c", subcore_axis_name="s"),
    out_type=jax.ShapeDtypeStruct((TOTAL, EMBED_DIM), jnp.float32),
    scratch_types=[
        pltpu.VMEM((NUM_CHUNKS, CHUNK), jnp.int32),
        pltpu.VMEM((ROWS_PER_GROUP, EMBED_DIM), jnp.float32),
        pltpu.SemaphoreType.DMA,
        pltpu.SemaphoreType.DMA,
    ],
    compiler_params=pltpu.CompilerParams(use_tc_tiling_on_sc=False),
)
def _gather_kernel(idx_hbm, table_hbm, out_hbm, idx_v, rows_v, gsem, osem):
    wid = lax.axis_index("s") * NUM_CORES + lax.axis_index("c")
    base = wid * PER_WORKER
    pltpu.sync_copy(idx_hbm.at[wid], idx_v)

    def group_body(g, _):
        copies = []
        for j in range(GROUP):
            c = pltpu.async_copy(
                table_hbm.at[idx_v.at[g * GROUP + j]],
                rows_v.at[pl.ds(j * CHUNK, CHUNK)],
                gsem,
            )
            copies.append(c)
        for c in copies:
            c.wait()
        pltpu.async_copy(
            rows_v,
            out_hbm.at[pl.ds(base + g * ROWS_PER_GROUP, ROWS_PER_GROUP)],
            osem,
        ).wait()
        return ()

    lax.fori_loop(0, NUM_GROUPS, group_body, ())


def kernel(indices, weight):
    idx = indices.astype(jnp.int32)
    idx = (idx % V_QUARTER) * 4 + idx // V_QUARTER
    idx = idx.reshape(NUM_WORKERS, NUM_CHUNKS, CHUNK)
    table_lin = _repack(jnp.transpose(weight)).reshape(NODES_LIN, EMBED_DIM)
    out = _gather_kernel(idx, table_lin)
    return out.reshape(BATCH, FIELDS, EMBED_DIM)


# speedup vs baseline: 2.2110x; 1.0865x over previous
import functools

import jax
import jax.numpy as jnp
from jax import lax
from jax.experimental import pallas as pl
from jax.experimental.pallas import tpu as pltpu
from jax.experimental.pallas import tpu_sc as plsc

BATCH = 16384
FIELDS = 26
EMBED_DIM = 32
NUM_NODES = 1000000

NUM_CORES = 2
NUM_SUBCORES = 16
NUM_WORKERS = NUM_CORES * NUM_SUBCORES

TOTAL = BATCH * FIELDS
PER_WORKER = TOTAL // NUM_WORKERS
CHUNK = 128
GROUP = 13
ROWS_PER_GROUP = CHUNK * GROUP
NUM_GROUPS = PER_WORKER // ROWS_PER_GROUP
NUM_CHUNKS = PER_WORKER // CHUNK

RCOLS = 4096
RGRID = 62
V_QUARTER = RCOLS * RGRID
NODES_LIN = 4 * V_QUARTER

assert PER_WORKER * NUM_WORKERS == TOTAL
assert ROWS_PER_GROUP * NUM_GROUPS == PER_WORKER


def _repack_block(w0, w1, w2, w3, out_ref):
    out_ref[:, 0:32] = jnp.transpose(w0[...])
    out_ref[:, 32:64] = jnp.transpose(w1[...])
    out_ref[:, 64:96] = jnp.transpose(w2[...])
    out_ref[:, 96:128] = jnp.transpose(w3[...])


def _repack(wt):
    max_blk = NUM_NODES // RCOLS
    specs = [
        pl.BlockSpec(
            (EMBED_DIM, RCOLS),
            functools.partial(
                lambda q, i: (0, jnp.minimum(q * RGRID + i, max_blk)), q
            ),
        )
        for q in range(4)
    ]
    return pl.pallas_call(
        _repack_block,
        grid=(RGRID,),
        in_specs=specs,
        out_specs=pl.BlockSpec((RCOLS, 128), lambda i: (i, 0)),
        out_shape=jax.ShapeDtypeStruct((V_QUARTER, 128), jnp.float32),
    )(wt, wt, wt, wt)


@functools.partial(
    pl.kernel,
    mesh=plsc.VectorSubcoreMesh(core_axis_name="c", subcore_axis_name="s"),
    out_type=jax.ShapeDtypeStruct((TOTAL, EMBED_DIM), jnp.float32),
    scratch_types=[
        pltpu.VMEM((NUM_CHUNKS, CHUNK), jnp.int32),
        pltpu.VMEM((ROWS_PER_GROUP, EMBED_DIM), jnp.float32),
        pltpu.SemaphoreType.DMA,
        pltpu.SemaphoreType.DMA,
    ],
    compiler_params=pltpu.CompilerParams(use_tc_tiling_on_sc=False),
)
def _gather_kernel(idx_hbm, table_hbm, out_hbm, idx_v, rows_v, gsem, osem):
    wid = lax.axis_index("s") * NUM_CORES + lax.axis_index("c")
    base = wid * PER_WORKER
    pltpu.sync_copy(idx_hbm.at[wid], idx_v)

    def group_body(g, _):
        copies = []
        for j in range(GROUP):
            c = pltpu.async_copy(
                table_hbm.at[idx_v.at[g * GROUP + j]],
                rows_v.at[pl.ds(j * CHUNK, CHUNK)],
                gsem,
            )
            copies.append(c)
        for c in copies:
            c.wait()
        pltpu.async_copy(
            rows_v,
            out_hbm.at[pl.ds(base + g * ROWS_PER_GROUP, ROWS_PER_GROUP)],
            osem,
        ).wait()
        return ()

    lax.fori_loop(0, NUM_GROUPS, group_body, ())


def kernel(indices, weight):
    idx = indices.astype(jnp.int32)
    idx = (idx % V_QUARTER) * 4 + idx // V_QUARTER
    idx = idx.reshape(NUM_WORKERS, NUM_CHUNKS, CHUNK)
    table_lin = _repack(jnp.transpose(weight)).reshape(NODES_LIN, EMBED_DIM)
    out = _gather_kernel(idx, table_lin)
    return out.reshape(BATCH, FIELDS, EMBED_DIM)
